# baseline (speedup 1.0000x reference)
import jax
import jax.numpy as jnp
from jax.experimental import pallas as pl
from jax.experimental.pallas import tpu as pltpu

N_RETINA = 4096
N_LGN = 1024
MU_WTS = 2.5
ETA = 0.1

_R_BLK = 512
_L_BLK = 256


def _stage1(f_ref, w_ref, thr_ref, out_ref):
    x = jax.lax.dot_general(
        f_ref[...], w_ref[...], (((1,), (1,)), ((), ())),
        precision=jax.lax.Precision.HIGHEST,
        preferred_element_type=jnp.float32)
    out_ref[...] = (x > thr_ref[...]).astype(jnp.float32)


def _stage2(f_ref, w_ref, act_ref, copy_ref):
    x = jax.lax.dot_general(
        f_ref[...], w_ref[...], (((1,), (1,)), ((), ())),
        precision=jax.lax.Precision.HIGHEST,
        preferred_element_type=jnp.float32)
    act_ref[...] = jnp.maximum(x, 0.0)
    copy_ref[...] = w_ref[...]


def _stage3(act_ref, thr_ref, f_ref, w_in_ref, w_out_ref, thr_out_ref,
            row_ref, sem):
    del w_in_ref
    act = jnp.maximum(act_ref[...] - thr_ref[...], 0.0)
    max_val = jnp.max(act)
    idx = jax.lax.broadcasted_iota(jnp.int32, (1, N_LGN), 1)
    max_idx = jnp.min(jnp.where(act == max_val, idx, jnp.int32(N_LGN)))
    thr_out_ref[...] = thr_ref[...] + jnp.where(
        idx == max_idx, 0.005 * max_val, 0.0)

    @pl.when(max_val > 0.0)
    def _():
        cp_in = pltpu.make_async_copy(
            w_out_ref.at[pl.ds(max_idx, 1)], row_ref, sem)
        cp_in.start()
        cp_in.wait()
        row = row_ref[...] + (ETA * max_val) * f_ref[...]
        mean = jnp.sum(row) * (1.0 / N_RETINA)
        row_ref[...] = row / mean * MU_WTS
        cp_out = pltpu.make_async_copy(
            row_ref, w_out_ref.at[pl.ds(max_idx, 1)], sem)
        cp_out.start()
        cp_out.wait()


def kernel(is_firing, node_weights, node_threshold, lgn_weights, lgn_threshold):
    f0 = is_firing.reshape(1, N_RETINA)
    nthr = node_threshold.reshape(1, N_RETINA)
    lthr = lgn_threshold.reshape(1, N_LGN)

    firing = pl.pallas_call(
        _stage1,
        grid=(N_RETINA // _R_BLK,),
        in_specs=[
            pl.BlockSpec((1, N_RETINA), lambda i: (0, 0)),
            pl.BlockSpec((_R_BLK, N_RETINA), lambda i: (i, 0)),
            pl.BlockSpec((1, _R_BLK), lambda i: (0, i)),
        ],
        out_specs=pl.BlockSpec((1, _R_BLK), lambda i: (0, i)),
        out_shape=jax.ShapeDtypeStruct((1, N_RETINA), jnp.float32),
    )(f0, node_weights, nthr)

    act_raw, w_copy = pl.pallas_call(
        _stage2,
        grid=(N_LGN // _L_BLK,),
        in_specs=[
            pl.BlockSpec((1, N_RETINA), lambda i: (0, 0)),
            pl.BlockSpec((_L_BLK, N_RETINA), lambda i: (i, 0)),
        ],
        out_specs=[
            pl.BlockSpec((1, _L_BLK), lambda i: (0, i)),
            pl.BlockSpec((_L_BLK, N_RETINA), lambda i: (i, 0)),
        ],
        out_shape=[
            jax.ShapeDtypeStruct((1, N_LGN), jnp.float32),
            jax.ShapeDtypeStruct((N_LGN, N_RETINA), jnp.float32),
        ],
    )(firing, lgn_weights)

    new_w, new_thr = pl.pallas_call(
        _stage3,
        in_specs=[
            pl.BlockSpec((1, N_LGN), lambda: (0, 0)),
            pl.BlockSpec((1, N_LGN), lambda: (0, 0)),
            pl.BlockSpec((1, N_RETINA), lambda: (0, 0)),
            pl.BlockSpec(memory_space=pl.ANY),
        ],
        out_specs=[
            pl.BlockSpec(memory_space=pl.ANY),
            pl.BlockSpec((1, N_LGN), lambda: (0, 0)),
        ],
        out_shape=[
            jax.ShapeDtypeStruct((N_LGN, N_RETINA), jnp.float32),
            jax.ShapeDtypeStruct((1, N_LGN), jnp.float32),
        ],
        scratch_shapes=[
            pltpu.VMEM((1, N_RETINA), jnp.float32),
            pltpu.SemaphoreType.DMA,
        ],
        input_output_aliases={3: 0},
    )(act_raw, lthr, firing, w_copy)

    return (firing.reshape(N_RETINA), act_raw.reshape(N_LGN),
            new_w, new_thr.reshape(N_LGN))


# speedup vs baseline: 2.1419x; 2.1419x over previous
import jax
import jax.numpy as jnp
from jax.experimental import pallas as pl
from jax.experimental.pallas import tpu as pltpu

N_RETINA = 4096
N_LGN = 1024
MU_WTS = 2.5
ETA = 0.1

_R_BLK = 512
_L_BLK = 256
_N_STEP1 = N_RETINA // _R_BLK
_N_STEP2 = N_LGN // _L_BLK


def _mega(f_ref, nw_ref, nthr_ref, lw_ref, lthr_ref,
          fir_out_ref, act_ref, w_out_ref, thr_out_ref,
          fir_ref, row_ref, sem):
    i = pl.program_id(0)

    @pl.when(i < _N_STEP1)
    def _stage1():
        x = jnp.sum(nw_ref[...] * f_ref[...], axis=1, keepdims=True)
        xr = jnp.transpose(x, (1, 0))
        bits = (xr > nthr_ref[...]).astype(jnp.float32)
        fir_out_ref[...] = bits
        fir_ref[:, pl.ds(i * _R_BLK, _R_BLK)] = bits

    @pl.when((i >= _N_STEP1) & (i < _N_STEP1 + _N_STEP2))
    def _stage2():
        j = i - _N_STEP1
        cp = pltpu.make_async_copy(
            lw_ref, w_out_ref.at[pl.ds(j * _L_BLK, _L_BLK)], sem)
        cp.start()
        a = jnp.sum(lw_ref[...] * fir_ref[...], axis=1, keepdims=True)
        ar = jnp.transpose(a, (1, 0))
        act_ref[:, pl.ds(j * _L_BLK, _L_BLK)] = jnp.maximum(ar, 0.0)
        cp.wait()

    @pl.when(i == _N_STEP1 + _N_STEP2)
    def _stage3():
        act = jnp.maximum(act_ref[...] - lthr_ref[...], 0.0)
        max_val = jnp.max(act)
        idx = jax.lax.broadcasted_iota(jnp.int32, (1, N_LGN), 1)
        max_idx = jnp.min(jnp.where(act == max_val, idx, jnp.int32(N_LGN)))
        thr_out_ref[...] = lthr_ref[...] + jnp.where(
            idx == max_idx, 0.005 * max_val, 0.0)

        @pl.when(max_val > 0.0)
        def _():
            cp_in = pltpu.make_async_copy(
                w_out_ref.at[pl.ds(max_idx, 1)], row_ref, sem)
            cp_in.start()
            cp_in.wait()
            row = row_ref[...] + (ETA * max_val) * fir_ref[...]
            mean = jnp.sum(row) * (1.0 / N_RETINA)
            row_ref[...] = row / mean * MU_WTS
            cp_out = pltpu.make_async_copy(
                row_ref, w_out_ref.at[pl.ds(max_idx, 1)], sem)
            cp_out.start()
            cp_out.wait()


def kernel(is_firing, node_weights, node_threshold, lgn_weights, lgn_threshold):
    f0 = is_firing.reshape(1, N_RETINA)
    nthr = node_threshold.reshape(1, N_RETINA)
    lthr = lgn_threshold.reshape(1, N_LGN)
    n_steps = _N_STEP1 + _N_STEP2 + 1

    firing, act_raw, new_w, new_thr = pl.pallas_call(
        _mega,
        grid=(n_steps,),
        in_specs=[
            pl.BlockSpec((1, N_RETINA), lambda i: (0, 0)),
            pl.BlockSpec((_R_BLK, N_RETINA),
                         lambda i: (jnp.minimum(i, _N_STEP1 - 1), 0)),
            pl.BlockSpec((1, _R_BLK),
                         lambda i: (0, jnp.minimum(i, _N_STEP1 - 1))),
            pl.BlockSpec((_L_BLK, N_RETINA),
                         lambda i: (jnp.clip(i - _N_STEP1, 0, _N_STEP2 - 1), 0)),
            pl.BlockSpec((1, N_LGN), lambda i: (0, 0)),
        ],
        out_specs=[
            pl.BlockSpec((1, _R_BLK),
                         lambda i: (0, jnp.minimum(i, _N_STEP1 - 1))),
            pl.BlockSpec((1, N_LGN), lambda i: (0, 0)),
            pl.BlockSpec(memory_space=pl.ANY),
            pl.BlockSpec((1, N_LGN), lambda i: (0, 0)),
        ],
        out_shape=[
            jax.ShapeDtypeStruct((1, N_RETINA), jnp.float32),
            jax.ShapeDtypeStruct((1, N_LGN), jnp.float32),
            jax.ShapeDtypeStruct((N_LGN, N_RETINA), jnp.float32),
            jax.ShapeDtypeStruct((1, N_LGN), jnp.float32),
        ],
        scratch_shapes=[
            pltpu.VMEM((1, N_RETINA), jnp.float32),
            pltpu.VMEM((1, N_RETINA), jnp.float32),
            pltpu.SemaphoreType.DMA,
        ],
    )(f0, node_weights, nthr, lgn_weights, lthr)

    return (firing.reshape(N_RETINA), act_raw.reshape(N_LGN),
            new_w, new_thr.reshape(N_LGN))
